# baseline (speedup 1.0000x reference)
import jax
import jax.numpy as jnp
from jax import lax
from jax.experimental import pallas as pl
from jax.experimental.pallas import tpu as pltpu
from jax.experimental.pallas import tpu_sc as plsc

N_NODES = 10000
FEAT = 128
N_EDGES = 320000

NC = 2
NS = 16
N_PAD = 10240
E_PAD = 327680
CHUNK = 128
TILE_EDGES = E_PAD // (NC * NS)
TILE_CHUNKS = TILE_EDGES // CHUNK
TILE_ROWS = N_PAD // NS
DUMMY = N_NODES

_MESH = plsc.VectorSubcoreMesh(core_axis_name="c", subcore_axis_name="s")


_DEPTH = 8


def _deg_body(esd_hbm, ones_hbm, z1_hbm, deg_out, didx, ones_v, deg_sp, ssem):
    c = lax.axis_index("c")
    s = lax.axis_index("s")
    cbase = (c * NS + s) * TILE_CHUNKS
    pltpu.sync_copy(ones_hbm, ones_v)
    pltpu.sync_copy(esd_hbm.at[pl.ds(cbase, TILE_CHUNKS)], didx)
    pltpu.sync_copy(z1_hbm.at[pl.ds(s * TILE_ROWS, TILE_ROWS)],
                    deg_sp.at[pl.ds(s * TILE_ROWS, TILE_ROWS)])
    plsc.subcore_barrier()

    def fire(ci):
        pltpu.async_copy(ones_v, deg_sp.at[didx.at[ci, 1]], ssem, add=True)

    def drain():
        pltpu.make_async_copy(ones_v, deg_sp.at[didx.at[0, 1]], ssem).wait()

    for ci in range(_DEPTH):
        fire(ci)

    def step(g, carry):
        drain()
        fire(g + _DEPTH)
        return carry

    lax.fori_loop(0, TILE_CHUNKS - _DEPTH, step, 0)
    for _ in range(_DEPTH):
        drain()
    plsc.subcore_barrier()
    pltpu.sync_copy(deg_sp.at[pl.ds(s * TILE_ROWS, TILE_ROWS)],
                    deg_out.at[c, pl.ds(s * TILE_ROWS, TILE_ROWS)])


def _deg_kernel(esd, ones1, z1):
    return pl.kernel(
        _deg_body,
        out_type=jax.ShapeDtypeStruct((NC, N_PAD), jnp.float32),
        mesh=_MESH,
        scratch_types=[
            pltpu.VMEM((TILE_CHUNKS, 2, CHUNK), jnp.int32),
            pltpu.VMEM((CHUNK,), jnp.float32),
            pltpu.VMEM_SHARED((N_PAD,), jnp.float32),
            pltpu.SemaphoreType.DMA,
        ],
    )(esd, ones1, z1)


def _agg_body(hp_hbm, esd_hbm, acc_out,
              sdidx, rows, acc_sp, isem, gsem, ssem):
    c = lax.axis_index("c")
    s = lax.axis_index("s")
    cbase = (c * NS + s) * TILE_CHUNKS
    pltpu.sync_copy(hp_hbm.at[pl.ds(s * TILE_ROWS, TILE_ROWS)],
                    acc_sp.at[pl.ds(s * TILE_ROWS, TILE_ROWS)])
    plsc.subcore_barrier()

    def fire_i(ci, isl):
        pltpu.async_copy(esd_hbm.at[cbase + ci], sdidx.at[isl], isem)

    def wait_i(ci, isl):
        pltpu.make_async_copy(esd_hbm.at[cbase + ci], sdidx.at[isl],
                              isem).wait()

    def fire_g(b, isl):
        pltpu.async_copy(hp_hbm.at[sdidx.at[isl, 0]], rows.at[b], gsem)

    def wait_g(b, isl):
        pltpu.make_async_copy(hp_hbm.at[sdidx.at[isl, 0]], rows.at[b],
                              gsem).wait()

    def scat(b, isl):
        pltpu.async_copy(rows.at[b], acc_sp.at[sdidx.at[isl, 1]], ssem,
                         add=True)

    def wait_s(b, isl):
        pltpu.make_async_copy(rows.at[b], acc_sp.at[sdidx.at[isl, 1]],
                              ssem).wait()

    def process(ci, k, fire_idx, fire_gather, wait_prev):
        b = k % 2
        wait_g(b, k)
        scat(b, k)
        if wait_prev:
            wait_s(1 - b, (k + 3) % 4)
        if fire_idx:
            fire_i(ci + 3, (k + 3) % 4)
        if fire_gather:
            wait_i(ci + 1, (k + 1) % 4)
            fire_g(1 - b, (k + 1) % 4)

    fire_i(0, 0)
    fire_i(1, 1)
    fire_i(2, 2)
    wait_i(0, 0)
    fire_g(0, 0)
    process(0, 0, True, True, False)
    for k in range(1, 4):
        process(k, k, True, True, True)

    def outer(g, carry):
        for k in range(4):
            process(4 * g + k, k, True, True, True)
        return carry

    lax.fori_loop(1, TILE_CHUNKS // 4 - 1, outer, 0)
    base = TILE_CHUNKS - 4
    process(base + 0, 0, True, True, True)
    process(base + 1, 1, False, True, True)
    process(base + 2, 2, False, True, True)
    process(base + 3, 3, False, False, True)
    wait_s(1, 3)
    plsc.subcore_barrier()
    pltpu.sync_copy(acc_sp.at[pl.ds(s * TILE_ROWS, TILE_ROWS)],
                    acc_out.at[c, pl.ds(s * TILE_ROWS, TILE_ROWS)])


def _agg_kernel(hp, esd):
    return pl.kernel(
        _agg_body,
        out_type=jax.ShapeDtypeStruct((NC, N_PAD, FEAT), jnp.float32),
        mesh=_MESH,
        scratch_types=[
            pltpu.VMEM((4, 2, CHUNK), jnp.int32),
            pltpu.VMEM((2, CHUNK, FEAT), jnp.float32),
            pltpu.VMEM_SHARED((N_PAD, FEAT), jnp.float32),
            pltpu.SemaphoreType.DMA,
            pltpu.SemaphoreType.DMA,
            pltpu.SemaphoreType.DMA,
        ],
    )(hp, esd)


def _agg2_body(sp_hbm, esd_hbm, z1_hbm, agg_out,
               sdall, vals, acc_sp, sp_sp, sem, ssem):
    c = lax.axis_index("c")
    s = lax.axis_index("s")
    cbase = (c * NS + s) * TILE_CHUNKS
    pltpu.sync_copy(esd_hbm.at[pl.ds(cbase, TILE_CHUNKS)], sdall)
    pltpu.sync_copy(z1_hbm.at[pl.ds(s * TILE_ROWS, TILE_ROWS)],
                    acc_sp.at[pl.ds(s * TILE_ROWS, TILE_ROWS)])
    pltpu.sync_copy(sp_hbm.at[pl.ds(s * TILE_ROWS, TILE_ROWS)],
                    sp_sp.at[pl.ds(s * TILE_ROWS, TILE_ROWS)])
    plsc.subcore_barrier()

    def fire_g(ci):
        pltpu.async_copy(sp_sp.at[sdall.at[ci, 0]], vals.at[ci], sem)

    def wait_g(ci):
        pltpu.make_async_copy(sp_sp.at[sdall.at[ci, 0]], vals.at[ci],
                              sem).wait()

    def fire_s(ci):
        pltpu.async_copy(vals.at[ci], acc_sp.at[sdall.at[ci, 1]], ssem,
                         add=True)

    def wait_s(ci):
        pltpu.make_async_copy(vals.at[ci], acc_sp.at[sdall.at[ci, 1]],
                              ssem).wait()

    for ci in range(_DEPTH):
        fire_g(ci)

    def step(g, carry):
        wait_g(g)
        fire_s(g)
        fire_g(g + _DEPTH)
        return carry

    lax.fori_loop(0, TILE_CHUNKS - _DEPTH, step, 0)
    for ci in range(TILE_CHUNKS - _DEPTH, TILE_CHUNKS):
        wait_g(ci)
        fire_s(ci)
    lax.fori_loop(0, TILE_CHUNKS, lambda g, cr: (wait_s(g), cr)[1], 0)
    plsc.subcore_barrier()
    pltpu.sync_copy(acc_sp.at[pl.ds(s * TILE_ROWS, TILE_ROWS)],
                    agg_out.at[c, pl.ds(s * TILE_ROWS, TILE_ROWS)])


def _agg2_kernel(sp, esd, z1):
    return pl.kernel(
        _agg2_body,
        out_type=jax.ShapeDtypeStruct((NC, N_PAD), jnp.float32),
        mesh=_MESH,
        scratch_types=[
            pltpu.VMEM((TILE_CHUNKS, 2, CHUNK), jnp.int32),
            pltpu.VMEM((TILE_CHUNKS, CHUNK), jnp.float32),
            pltpu.VMEM_SHARED((N_PAD,), jnp.float32),
            pltpu.VMEM_SHARED((N_PAD,), jnp.float32),
            pltpu.SemaphoreType.DMA,
            pltpu.SemaphoreType.DMA,
        ],
    )(sp, esd, z1)


_BLK = 1024
_NBLK = N_PAD // _BLK


def _mmh_body(x_ref, w_ref, h_ref):
    h_ref[...] = jnp.dot(x_ref[...], w_ref[...],
                         preferred_element_type=jnp.float32)


def _mmh_kernel(x_pad, W1):
    return pl.pallas_call(
        _mmh_body,
        grid=(_NBLK,),
        in_specs=[
            pl.BlockSpec((_BLK, FEAT), lambda i: (i, 0)),
            pl.BlockSpec((FEAT, FEAT), lambda i: (0, 0)),
        ],
        out_specs=pl.BlockSpec((_BLK, FEAT), lambda i: (i, 0)),
        out_shape=jax.ShapeDtypeStruct((N_PAD, FEAT), jnp.float32),
    )(x_pad, W1)


def _scale_body(h_ref, degp_ref, hp_ref, dinv_ref):
    deg = degp_ref[0] + degp_ref[1] + 1.0
    dinv = lax.rsqrt(deg)
    hp_ref[...] = h_ref[...] * dinv
    dinv_ref[...] = dinv


def _scale_kernel(h, degp3):
    return pl.pallas_call(
        _scale_body,
        grid=(_NBLK,),
        in_specs=[
            pl.BlockSpec((_BLK, FEAT), lambda i: (i, 0)),
            pl.BlockSpec((NC, _BLK, 1), lambda i: (0, i, 0)),
        ],
        out_specs=[
            pl.BlockSpec((_BLK, FEAT), lambda i: (i, 0)),
            pl.BlockSpec((_BLK, 1), lambda i: (i, 0)),
        ],
        out_shape=[
            jax.ShapeDtypeStruct((N_PAD, FEAT), jnp.float32),
            jax.ShapeDtypeStruct((N_PAD, 1), jnp.float32),
        ],
    )(h, degp3)


def _post_body(acc_ref, hp_ref, dinv_ref, b1_ref, gamma_ref, beta_ref,
               w2_ref, sp_ref, out1_ref, stats_ref):
    p = pl.program_id(0)
    i = pl.program_id(1)
    rows = lax.broadcasted_iota(jnp.int32, (_BLK, 1), 0) + i * _BLK
    mask = rows < N_NODES

    @pl.when(p == 0)
    def _():
        val = ((acc_ref[0] + acc_ref[1] - hp_ref[...]) * dinv_ref[...]
               + b1_ref[...])
        val = jnp.where(mask, val, 0.0)
        out1_ref[pl.ds(i * _BLK, _BLK), :] = val

        @pl.when(i == 0)
        def _():
            stats_ref[...] = jnp.zeros_like(stats_ref)

        stats_ref[0:1, :] += jnp.sum(val, axis=0, keepdims=True)
        stats_ref[1:2, :] += jnp.sum(val * val, axis=0, keepdims=True)
        sp_ref[...] = jnp.zeros_like(sp_ref)

    @pl.when(p == 1)
    def _():
        inv_n = 1.0 / N_NODES
        mean = stats_ref[0:1, :] * inv_n
        var = stats_ref[1:2, :] * inv_n - mean * mean
        rstd = lax.rsqrt(var + 1e-5)
        xb = out1_ref[pl.ds(i * _BLK, _BLK), :]
        bn = (xb - mean) * (rstd * gamma_ref[...]) + beta_ref[...]
        r = jnp.maximum(bn, 0.0)
        sv = jnp.dot(r, w2_ref[...], preferred_element_type=jnp.float32)
        sp_ref[...] = jnp.where(mask, sv * dinv_ref[...], 0.0)


def _post_kernel(accp, hp, dinv, b1r, gammar, betar, W2):
    return pl.pallas_call(
        _post_body,
        grid=(2, _NBLK),
        in_specs=[
            pl.BlockSpec((NC, _BLK, FEAT), lambda p, i: (0, i * (1 - p), 0)),
            pl.BlockSpec((_BLK, FEAT), lambda p, i: (i * (1 - p), 0)),
            pl.BlockSpec((_BLK, 1), lambda p, i: (i, 0)),
            pl.BlockSpec((1, FEAT), lambda p, i: (0, 0)),
            pl.BlockSpec((1, FEAT), lambda p, i: (0, 0)),
            pl.BlockSpec((1, FEAT), lambda p, i: (0, 0)),
            pl.BlockSpec((FEAT, 1), lambda p, i: (0, 0)),
        ],
        out_specs=pl.BlockSpec((_BLK, 1), lambda p, i: (i, 0)),
        out_shape=jax.ShapeDtypeStruct((N_PAD, 1), jnp.float32),
        scratch_shapes=[
            pltpu.VMEM((N_PAD, FEAT), jnp.float32),
            pltpu.VMEM((8, FEAT), jnp.float32),
        ],
    )(accp, hp, dinv, b1r, gammar, betar, W2)


def _fin_body(agg_ref, sp_ref, dinv_ref, b2_ref, out_ref):
    out_ref[...] = ((agg_ref[0] + agg_ref[1] + sp_ref[...]) * dinv_ref[...]
                    + b2_ref[...])


def _fin_kernel(agg3, sp, dinv, b2r):
    return pl.pallas_call(
        _fin_body,
        grid=(_NBLK,),
        in_specs=[
            pl.BlockSpec((NC, _BLK, 1), lambda i: (0, i, 0)),
            pl.BlockSpec((_BLK, 1), lambda i: (i, 0)),
            pl.BlockSpec((_BLK, 1), lambda i: (i, 0)),
            pl.BlockSpec((1, 1), lambda i: (0, 0)),
        ],
        out_specs=pl.BlockSpec((_BLK, 1), lambda i: (i, 0)),
        out_shape=jax.ShapeDtypeStruct((N_PAD, 1), jnp.float32),
    )(agg3, sp, dinv, b2r)


def kernel(x, edge_index, W1, b1, gamma, beta, W2, b2):
    ei = edge_index.astype(jnp.int32)
    epad = DUMMY + jnp.arange(E_PAD - N_EDGES, dtype=jnp.int32) % (
        N_PAD - N_NODES)
    src2 = jnp.concatenate([ei[0], epad]).reshape(E_PAD // CHUNK, CHUNK)
    dst2 = jnp.concatenate([ei[1], epad]).reshape(E_PAD // CHUNK, CHUNK)
    esd = jnp.stack([src2, dst2], axis=1)
    del src2, dst2
    x_pad = jnp.concatenate(
        [x, jnp.zeros((N_PAD - N_NODES, FEAT), jnp.float32)])
    z1 = jnp.zeros((N_PAD,), jnp.float32)
    ones1 = jnp.ones((CHUNK,), jnp.float32)

    h = _mmh_kernel(x_pad, W1)
    degp = _deg_kernel(esd, ones1, z1)
    hp, dinv = _scale_kernel(h, degp.reshape(NC, N_PAD, 1))
    accp = _agg_kernel(hp, esd)
    sp = _post_kernel(accp, hp, dinv, b1.reshape(1, FEAT),
                      gamma.reshape(1, FEAT), beta.reshape(1, FEAT),
                      W2)
    agg2 = _agg2_kernel(sp.reshape(N_PAD), esd, z1)
    out2 = _fin_kernel(agg2.reshape(NC, N_PAD, 1), sp, dinv,
                       b2.reshape(1, 1))
    return out2[:N_NODES]


# speedup vs baseline: 1.0291x; 1.0291x over previous
import jax
import jax.numpy as jnp
from jax import lax
from jax.experimental import pallas as pl
from jax.experimental.pallas import tpu as pltpu
from jax.experimental.pallas import tpu_sc as plsc

N_NODES = 10000
FEAT = 128
N_EDGES = 320000

NC = 2
NS = 16
N_PAD = 10240
E_PAD = 327680
CHUNK = 128
TILE_EDGES = E_PAD // (NC * NS)
TILE_CHUNKS = TILE_EDGES // CHUNK
TILE_ROWS = N_PAD // NS
DUMMY = N_NODES

_MESH = plsc.VectorSubcoreMesh(core_axis_name="c", subcore_axis_name="s")


_DEPTH = 8


def _deg_body(esd_hbm, deg_out, didx, ones_v, zrow, deg_sp, ssem):
    c = lax.axis_index("c")
    s = lax.axis_index("s")
    cbase = (c * NS + s) * TILE_CHUNKS
    pltpu.sync_copy(esd_hbm.at[pl.ds(cbase, TILE_CHUNKS)], didx)
    for i in range(CHUNK // 16):
        ones_v[pl.ds(16 * i, 16)] = jnp.ones((16,), jnp.float32)
    for i in range(TILE_ROWS // 16):
        zrow[pl.ds(16 * i, 16)] = jnp.zeros((16,), jnp.float32)
    pltpu.sync_copy(zrow, deg_sp.at[pl.ds(s * TILE_ROWS, TILE_ROWS)])
    plsc.subcore_barrier()

    def fire(ci):
        pltpu.async_copy(ones_v, deg_sp.at[didx.at[ci, 1]], ssem, add=True)

    def drain():
        pltpu.make_async_copy(ones_v, deg_sp.at[didx.at[0, 1]], ssem).wait()

    for ci in range(_DEPTH):
        fire(ci)

    def step(g, carry):
        drain()
        fire(g + _DEPTH)
        return carry

    lax.fori_loop(0, TILE_CHUNKS - _DEPTH, step, 0)
    for _ in range(_DEPTH):
        drain()
    plsc.subcore_barrier()
    pltpu.sync_copy(deg_sp.at[pl.ds(s * TILE_ROWS, TILE_ROWS)],
                    deg_out.at[c, pl.ds(s * TILE_ROWS, TILE_ROWS)])


def _deg_kernel(esd):
    return pl.kernel(
        _deg_body,
        out_type=jax.ShapeDtypeStruct((NC, N_PAD), jnp.float32),
        mesh=_MESH,
        scratch_types=[
            pltpu.VMEM((TILE_CHUNKS, 2, CHUNK), jnp.int32),
            pltpu.VMEM((CHUNK,), jnp.float32),
            pltpu.VMEM((TILE_ROWS,), jnp.float32),
            pltpu.VMEM_SHARED((N_PAD,), jnp.float32),
            pltpu.SemaphoreType.DMA,
        ],
    )(esd)


def _agg_body(hp_hbm, esd_hbm, acc_out,
              sdidx, rows, acc_sp, isem, gsem, ssem):
    c = lax.axis_index("c")
    s = lax.axis_index("s")
    cbase = (c * NS + s) * TILE_CHUNKS
    pltpu.sync_copy(hp_hbm.at[pl.ds(s * TILE_ROWS, TILE_ROWS)],
                    acc_sp.at[pl.ds(s * TILE_ROWS, TILE_ROWS)])
    plsc.subcore_barrier()

    def fire_i(ci, isl):
        pltpu.async_copy(esd_hbm.at[cbase + ci], sdidx.at[isl], isem)

    def wait_i(ci, isl):
        pltpu.make_async_copy(esd_hbm.at[cbase + ci], sdidx.at[isl],
                              isem).wait()

    def fire_g(b, isl):
        pltpu.async_copy(hp_hbm.at[sdidx.at[isl, 0]], rows.at[b], gsem)

    def wait_g(b, isl):
        pltpu.make_async_copy(hp_hbm.at[sdidx.at[isl, 0]], rows.at[b],
                              gsem).wait()

    def scat(b, isl):
        pltpu.async_copy(rows.at[b], acc_sp.at[sdidx.at[isl, 1]], ssem,
                         add=True)

    def wait_s(b, isl):
        pltpu.make_async_copy(rows.at[b], acc_sp.at[sdidx.at[isl, 1]],
                              ssem).wait()

    def process(ci, k, fire_idx, fire_gather, wait_prev):
        b = k % 2
        wait_g(b, k)
        scat(b, k)
        if wait_prev:
            wait_s(1 - b, (k + 3) % 4)
        if fire_idx:
            fire_i(ci + 3, (k + 3) % 4)
        if fire_gather:
            wait_i(ci + 1, (k + 1) % 4)
            fire_g(1 - b, (k + 1) % 4)

    fire_i(0, 0)
    fire_i(1, 1)
    fire_i(2, 2)
    wait_i(0, 0)
    fire_g(0, 0)
    process(0, 0, True, True, False)
    for k in range(1, 4):
        process(k, k, True, True, True)

    def outer(g, carry):
        for k in range(4):
            process(4 * g + k, k, True, True, True)
        return carry

    lax.fori_loop(1, TILE_CHUNKS // 4 - 1, outer, 0)
    base = TILE_CHUNKS - 4
    process(base + 0, 0, True, True, True)
    process(base + 1, 1, False, True, True)
    process(base + 2, 2, False, True, True)
    process(base + 3, 3, False, False, True)
    wait_s(1, 3)
    plsc.subcore_barrier()
    pltpu.sync_copy(acc_sp.at[pl.ds(s * TILE_ROWS, TILE_ROWS)],
                    acc_out.at[c, pl.ds(s * TILE_ROWS, TILE_ROWS)])


def _agg_kernel(hp, esd):
    return pl.kernel(
        _agg_body,
        out_type=jax.ShapeDtypeStruct((NC, N_PAD, FEAT), jnp.float32),
        mesh=_MESH,
        scratch_types=[
            pltpu.VMEM((4, 2, CHUNK), jnp.int32),
            pltpu.VMEM((2, CHUNK, FEAT), jnp.float32),
            pltpu.VMEM_SHARED((N_PAD, FEAT), jnp.float32),
            pltpu.SemaphoreType.DMA,
            pltpu.SemaphoreType.DMA,
            pltpu.SemaphoreType.DMA,
        ],
    )(hp, esd)


def _agg2_body(sp_hbm, esd_hbm, agg_out,
               sdall, vals, acc_sp, sp_sp, zrow, sem, ssem):
    c = lax.axis_index("c")
    s = lax.axis_index("s")
    cbase = (c * NS + s) * TILE_CHUNKS
    pltpu.sync_copy(esd_hbm.at[pl.ds(cbase, TILE_CHUNKS)], sdall)
    for i in range(TILE_ROWS // 16):
        zrow[pl.ds(16 * i, 16)] = jnp.zeros((16,), jnp.float32)
    pltpu.sync_copy(zrow, acc_sp.at[pl.ds(s * TILE_ROWS, TILE_ROWS)])
    pltpu.sync_copy(sp_hbm.at[pl.ds(s * TILE_ROWS, TILE_ROWS)],
                    sp_sp.at[pl.ds(s * TILE_ROWS, TILE_ROWS)])
    plsc.subcore_barrier()

    def fire_g(ci):
        pltpu.async_copy(sp_sp.at[sdall.at[ci, 0]], vals.at[ci], sem)

    def wait_g(ci):
        pltpu.make_async_copy(sp_sp.at[sdall.at[ci, 0]], vals.at[ci],
                              sem).wait()

    def fire_s(ci):
        pltpu.async_copy(vals.at[ci], acc_sp.at[sdall.at[ci, 1]], ssem,
                         add=True)

    def wait_s(ci):
        pltpu.make_async_copy(vals.at[ci], acc_sp.at[sdall.at[ci, 1]],
                              ssem).wait()

    for ci in range(_DEPTH):
        fire_g(ci)

    def step(g, carry):
        wait_g(g)
        fire_s(g)
        fire_g(g + _DEPTH)
        return carry

    lax.fori_loop(0, TILE_CHUNKS - _DEPTH, step, 0)
    for ci in range(TILE_CHUNKS - _DEPTH, TILE_CHUNKS):
        wait_g(ci)
        fire_s(ci)
    lax.fori_loop(0, TILE_CHUNKS, lambda g, cr: (wait_s(g), cr)[1], 0)
    plsc.subcore_barrier()
    pltpu.sync_copy(acc_sp.at[pl.ds(s * TILE_ROWS, TILE_ROWS)],
                    agg_out.at[c, pl.ds(s * TILE_ROWS, TILE_ROWS)])


def _agg2_kernel(sp, esd):
    return pl.kernel(
        _agg2_body,
        out_type=jax.ShapeDtypeStruct((NC, N_PAD), jnp.float32),
        mesh=_MESH,
        scratch_types=[
            pltpu.VMEM((TILE_CHUNKS, 2, CHUNK), jnp.int32),
            pltpu.VMEM((TILE_CHUNKS, CHUNK), jnp.float32),
            pltpu.VMEM_SHARED((N_PAD,), jnp.float32),
            pltpu.VMEM_SHARED((N_PAD,), jnp.float32),
            pltpu.VMEM((TILE_ROWS,), jnp.float32),
            pltpu.SemaphoreType.DMA,
            pltpu.SemaphoreType.DMA,
        ],
    )(sp, esd)


_BLK = 1024
_NBLK = N_PAD // _BLK


def _mm1_body(x_ref, w_ref, degp_ref, hp_ref, dinv_ref):
    deg = degp_ref[0] + degp_ref[1] + 1.0
    dinv = lax.rsqrt(deg)
    hp_ref[...] = jnp.dot(x_ref[...], w_ref[...],
                          preferred_element_type=jnp.float32) * dinv
    dinv_ref[...] = dinv


_XBLK = 1000


def _mm1_kernel(x, W1, degp3):
    return pl.pallas_call(
        _mm1_body,
        grid=(N_NODES // _XBLK,),
        in_specs=[
            pl.BlockSpec((_XBLK, FEAT), lambda i: (i, 0)),
            pl.BlockSpec((FEAT, FEAT), lambda i: (0, 0)),
            pl.BlockSpec((NC, _XBLK, 1), lambda i: (0, i, 0)),
        ],
        out_specs=[
            pl.BlockSpec((_XBLK, FEAT), lambda i: (i, 0)),
            pl.BlockSpec((_XBLK, 1), lambda i: (i, 0)),
        ],
        out_shape=[
            jax.ShapeDtypeStruct((N_PAD, FEAT), jnp.float32),
            jax.ShapeDtypeStruct((N_PAD, 1), jnp.float32),
        ],
    )(x, W1, degp3)


def _post_body(acc_ref, hp_ref, dinv_ref, b1_ref, gamma_ref, beta_ref,
               w2_ref, sp_ref, out1_ref, stats_ref):
    p = pl.program_id(0)
    i = pl.program_id(1)
    rows = lax.broadcasted_iota(jnp.int32, (_BLK, 1), 0) + i * _BLK
    mask = rows < N_NODES

    @pl.when(p == 0)
    def _():
        val = ((acc_ref[0] + acc_ref[1] - hp_ref[...]) * dinv_ref[...]
               + b1_ref[...])
        val = jnp.where(mask, val, 0.0)
        out1_ref[pl.ds(i * _BLK, _BLK), :] = val

        @pl.when(i == 0)
        def _():
            stats_ref[...] = jnp.zeros_like(stats_ref)

        stats_ref[0:1, :] += jnp.sum(val, axis=0, keepdims=True)
        stats_ref[1:2, :] += jnp.sum(val * val, axis=0, keepdims=True)
        sp_ref[...] = jnp.zeros_like(sp_ref)

    @pl.when(p == 1)
    def _():
        inv_n = 1.0 / N_NODES
        mean = stats_ref[0:1, :] * inv_n
        var = stats_ref[1:2, :] * inv_n - mean * mean
        rstd = lax.rsqrt(var + 1e-5)
        xb = out1_ref[pl.ds(i * _BLK, _BLK), :]
        bn = (xb - mean) * (rstd * gamma_ref[...]) + beta_ref[...]
        r = jnp.maximum(bn, 0.0)
        sv = jnp.dot(r, w2_ref[...], preferred_element_type=jnp.float32)
        sp_ref[...] = jnp.where(mask, sv * dinv_ref[...], 0.0)


def _post_kernel(accp, hp, dinv, b1r, gammar, betar, W2):
    return pl.pallas_call(
        _post_body,
        grid=(2, _NBLK),
        in_specs=[
            pl.BlockSpec((NC, _BLK, FEAT), lambda p, i: (0, i * (1 - p), 0)),
            pl.BlockSpec((_BLK, FEAT), lambda p, i: (i * (1 - p), 0)),
            pl.BlockSpec((_BLK, 1), lambda p, i: (i, 0)),
            pl.BlockSpec((1, FEAT), lambda p, i: (0, 0)),
            pl.BlockSpec((1, FEAT), lambda p, i: (0, 0)),
            pl.BlockSpec((1, FEAT), lambda p, i: (0, 0)),
            pl.BlockSpec((FEAT, 1), lambda p, i: (0, 0)),
        ],
        out_specs=pl.BlockSpec((_BLK, 1), lambda p, i: (i, 0)),
        out_shape=jax.ShapeDtypeStruct((N_PAD, 1), jnp.float32),
        scratch_shapes=[
            pltpu.VMEM((N_PAD, FEAT), jnp.float32),
            pltpu.VMEM((8, FEAT), jnp.float32),
        ],
    )(accp, hp, dinv, b1r, gammar, betar, W2)


def _fin_body(agg_ref, sp_ref, dinv_ref, b2_ref, out_ref):
    out_ref[...] = ((agg_ref[0] + agg_ref[1] + sp_ref[...]) * dinv_ref[...]
                    + b2_ref[...])


def _fin_kernel(agg3, sp, dinv, b2r):
    return pl.pallas_call(
        _fin_body,
        grid=(_NBLK,),
        in_specs=[
            pl.BlockSpec((NC, _BLK, 1), lambda i: (0, i, 0)),
            pl.BlockSpec((_BLK, 1), lambda i: (i, 0)),
            pl.BlockSpec((_BLK, 1), lambda i: (i, 0)),
            pl.BlockSpec((1, 1), lambda i: (0, 0)),
        ],
        out_specs=pl.BlockSpec((_BLK, 1), lambda i: (i, 0)),
        out_shape=jax.ShapeDtypeStruct((N_PAD, 1), jnp.float32),
    )(agg3, sp, dinv, b2r)


def kernel(x, edge_index, W1, b1, gamma, beta, W2, b2):
    ei = edge_index.astype(jnp.int32)
    epad = DUMMY + jnp.arange(E_PAD - N_EDGES, dtype=jnp.int32) % (
        N_PAD - N_NODES)
    src2 = jnp.concatenate([ei[0], epad]).reshape(E_PAD // CHUNK, CHUNK)
    dst2 = jnp.concatenate([ei[1], epad]).reshape(E_PAD // CHUNK, CHUNK)
    esd = jnp.stack([src2, dst2], axis=1)
    del src2, dst2
    degp = _deg_kernel(esd)
    hp, dinv = _mm1_kernel(x, W1, degp.reshape(NC, N_PAD, 1))
    accp = _agg_kernel(hp, esd)
    sp = _post_kernel(accp, hp, dinv, b1.reshape(1, FEAT),
                      gamma.reshape(1, FEAT), beta.reshape(1, FEAT),
                      W2)
    agg2 = _agg2_kernel(sp.reshape(N_PAD), esd)
    out2 = _fin_kernel(agg2.reshape(NC, N_PAD, 1), sp, dinv,
                       b2.reshape(1, 1))
    return out2[:N_NODES]
